# baseline (speedup 1.0000x reference)
import functools

import jax
import jax.numpy as jnp
from jax import lax
from jax.experimental import pallas as pl
from jax.experimental.pallas import tpu as pltpu
from jax.experimental.pallas import tpu_sc as plsc

N_NODES = 10000
N_EDGES = 320000
D_FEAT = 128

NC = 2
NS = 16
NW = NC * NS
LANES = 16

CHUNK = 128
K_CHUNKS = -(-N_EDGES // (NW * CHUNK))
K_CHUNKS += K_CHUNKS % 2
G = 8
NG = K_CHUNKS // G
K_IDX = K_CHUNKS + 2 * G
E_PAD = NW * K_CHUNKS * CHUNK
ACC_ROWS = 10240
STRIPE = ACC_ROWS // NS


def _sc_segment_sum(x, src3, dst3):
    mesh = plsc.VectorSubcoreMesh(core_axis_name="c", subcore_axis_name="s")

    @functools.partial(
        pl.kernel,
        mesh=mesh,
        out_type=jax.ShapeDtypeStruct((NC, ACC_ROWS, D_FEAT), jnp.float32),
        scratch_types=[
            pltpu.VMEM((K_IDX, CHUNK), jnp.int32),
            pltpu.VMEM((G, CHUNK), jnp.int32),
            pltpu.VMEM((G, CHUNK), jnp.int32),
            pltpu.VMEM((CHUNK, D_FEAT), jnp.float32),
            pltpu.VMEM((CHUNK, D_FEAT), jnp.float32),
            pltpu.VMEM_SHARED((ACC_ROWS, D_FEAT), jnp.float32),
            pltpu.SemaphoreType.DMA,
            pltpu.SemaphoreType.DMA,
            pltpu.SemaphoreType.DMA,
            pltpu.SemaphoreType.DMA,
        ],
    )
    def k(x_hbm, src_hbm, dst_hbm, out_hbm, sidx, dg0, dg1, rows0, rows1,
          acc, g0, g1, i0, i1):
        c = lax.axis_index("c")
        s = lax.axis_index("s")
        wid = s * NC + c

        def wait_dst(dg, isem):
            pltpu.make_async_copy(
                dst_hbm.at[wid, pl.ds(0, G)], dg, isem).wait()

        h_src = pltpu.async_copy(src_hbm.at[wid], sidx, g0)
        pltpu.async_copy(dst_hbm.at[wid, pl.ds(0, G)], dg0, i0)
        pltpu.async_copy(dst_hbm.at[wid, pl.ds(G, G)], dg1, i1)

        @pl.loop(0, CHUNK)
        def _(r):
            @pl.loop(0, D_FEAT, step=LANES)
            def _(col):
                rows0.at[pl.ds(r, 1), pl.ds(col, LANES)][...] = jnp.zeros(
                    (1, LANES), jnp.float32)

        base = s * STRIPE
        for b in range(STRIPE // CHUNK):
            pltpu.sync_copy(rows0, acc.at[pl.ds(base + b * CHUNK, CHUNK)])

        h_src.wait()
        plsc.subcore_barrier()

        pltpu.async_copy(x_hbm.at[sidx.at[0]], rows0, g0)

        @pl.loop(0, NG, step=2)
        def _(p):
            for gslot, dg, isem in ((0, dg0, i0), (1, dg1, i1)):
                grp = p + gslot
                jb = grp * G
                wait_dst(dg, isem)
                for t in range(G):
                    ch = jb + t
                    rb, rnb = (rows0, rows1) if t % 2 == 0 else (rows1, rows0)
                    sb, snb = (g0, g1) if t % 2 == 0 else (g1, g0)
                    pltpu.async_copy(x_hbm.at[sidx.at[ch + 1]], rnb, snb)
                    pltpu.make_async_copy(x_hbm.at[sidx.at[ch]], rb, sb).wait()
                    pltpu.sync_copy(rb, acc.at[dg.at[t]], add=True)
                pltpu.async_copy(
                    dst_hbm.at[wid, pl.ds((grp + 2) * G, G)], dg, isem)

        wait_dst(dg0, i0)
        wait_dst(dg1, i1)
        pltpu.make_async_copy(x_hbm.at[sidx.at[0]], rows0, g0).wait()

        plsc.subcore_barrier()

        pltpu.sync_copy(acc.at[pl.ds(base, STRIPE)],
                        out_hbm.at[c, pl.ds(base, STRIPE)])

    return k(x, src3, dst3)


def _combine(parts, W):
    blk = 1000

    def body(p_ref, w_ref, o_ref):
        o_ref[...] = (p_ref[0] + p_ref[1]) * w_ref[...]

    return pl.pallas_call(
        body,
        grid=(N_NODES // blk,),
        in_specs=[
            pl.BlockSpec((NC, blk, D_FEAT), lambda i: (0, i, 0)),
            pl.BlockSpec((1, D_FEAT), lambda i: (0, 0)),
        ],
        out_specs=pl.BlockSpec((blk, D_FEAT), lambda i: (i, 0)),
        out_shape=jax.ShapeDtypeStruct((N_NODES, D_FEAT), jnp.float32),
    )(parts, W)


def kernel(x, edge_index, W):
    src = edge_index[0]
    dst = edge_index[1]
    pad = E_PAD - N_EDGES
    src_p = jnp.concatenate([src, jnp.zeros((pad,), jnp.int32)])
    dst_p = jnp.concatenate([dst, jnp.full((pad,), N_NODES, jnp.int32)])
    ovr = K_IDX - K_CHUNKS
    src3 = jnp.concatenate(
        [src_p.reshape(NW, K_CHUNKS, CHUNK),
         jnp.zeros((NW, ovr, CHUNK), jnp.int32)], axis=1)
    dst3 = jnp.concatenate(
        [dst_p.reshape(NW, K_CHUNKS, CHUNK),
         jnp.full((NW, ovr, CHUNK), N_NODES, jnp.int32)], axis=1)
    parts = _sc_segment_sum(x, src3, dst3)
    return _combine(parts, W)


# speedup vs baseline: 1.1599x; 1.1599x over previous
import functools

import jax
import jax.numpy as jnp
from jax import lax
from jax.experimental import pallas as pl
from jax.experimental.pallas import tpu as pltpu
from jax.experimental.pallas import tpu_sc as plsc

N_NODES = 10000
N_EDGES = 320000
D_FEAT = 128

NC = 2
NS = 16
NW = NC * NS
LANES = 16

CHUNK = 128
K_CHUNKS = -(-N_EDGES // (NW * CHUNK))
K_CHUNKS += K_CHUNKS % 2
E_PAD = NW * K_CHUNKS * CHUNK
ACC_ROWS = 10240
STRIPE = ACC_ROWS // NS


def _sc_segment_sum(x, src3, dst3):
    mesh = plsc.VectorSubcoreMesh(core_axis_name="c", subcore_axis_name="s")

    @functools.partial(
        pl.kernel,
        mesh=mesh,
        out_type=jax.ShapeDtypeStruct((NC, ACC_ROWS, D_FEAT), jnp.float32),
        scratch_types=[
            pltpu.VMEM((K_CHUNKS, CHUNK), jnp.int32),
            pltpu.VMEM((K_CHUNKS, CHUNK), jnp.int32),
            pltpu.VMEM((CHUNK, D_FEAT), jnp.float32),
            pltpu.VMEM_SHARED((ACC_ROWS, D_FEAT), jnp.float32),
            pltpu.SemaphoreType.DMA,
        ],
    )
    def k(x_hbm, src_hbm, dst_hbm, out_hbm, sidx, didx, rows, acc, sem):
        c = lax.axis_index("c")
        s = lax.axis_index("s")
        wid = s * NC + c

        @pl.loop(0, CHUNK)
        def _(r):
            @pl.loop(0, D_FEAT, step=LANES)
            def _(col):
                rows.at[pl.ds(r, 1), pl.ds(col, LANES)][...] = jnp.zeros(
                    (1, LANES), jnp.float32)

        base = s * STRIPE
        for b in range(STRIPE // CHUNK):
            pltpu.sync_copy(rows, acc.at[pl.ds(base + b * CHUNK, CHUNK)])

        plsc.subcore_barrier()

        pltpu.sync_copy(src_hbm.at[wid], sidx)
        pltpu.sync_copy(dst_hbm.at[wid], didx)

        @pl.loop(0, K_CHUNKS)
        def _(j):
            pltpu.async_copy(x_hbm.at[sidx.at[j]], rows, sem).wait()
            pltpu.sync_copy(rows, acc.at[didx.at[j]], add=True)

        plsc.subcore_barrier()

        pltpu.sync_copy(acc.at[pl.ds(base, STRIPE)],
                        out_hbm.at[c, pl.ds(base, STRIPE)])

    return k(x, src3, dst3)


def _combine(parts, W):
    blk = 1000

    def body(p_ref, w_ref, o_ref):
        o_ref[...] = (p_ref[0] + p_ref[1]) * w_ref[...]

    return pl.pallas_call(
        body,
        grid=(N_NODES // blk,),
        in_specs=[
            pl.BlockSpec((NC, blk, D_FEAT), lambda i: (0, i, 0)),
            pl.BlockSpec((1, D_FEAT), lambda i: (0, 0)),
        ],
        out_specs=pl.BlockSpec((blk, D_FEAT), lambda i: (i, 0)),
        out_shape=jax.ShapeDtypeStruct((N_NODES, D_FEAT), jnp.float32),
    )(parts, W)


def kernel(x, edge_index, W):
    src = edge_index[0]
    dst = edge_index[1]
    pad = E_PAD - N_EDGES
    src3 = jnp.concatenate([src, jnp.zeros((pad,), jnp.int32)]).reshape(
        NW, K_CHUNKS, CHUNK)
    dst3 = jnp.concatenate([dst, jnp.full((pad,), N_NODES, jnp.int32)]).reshape(
        NW, K_CHUNKS, CHUNK)
    parts = _sc_segment_sum(x, src3, dst3)
    return _combine(parts, W)


# speedup vs baseline: 3.2503x; 2.8023x over previous
import functools

import jax
import jax.numpy as jnp
from jax import lax
from jax.experimental import pallas as pl
from jax.experimental.pallas import tpu as pltpu
from jax.experimental.pallas import tpu_sc as plsc

N_NODES = 10000
N_EDGES = 320000
D_FEAT = 128

NC = 2
NS = 16
NW = NC * NS
LANES = 16

CHUNK = 128
K_CHUNKS = -(-N_EDGES // (NW * CHUNK))
K_CHUNKS += K_CHUNKS % 2
E_PAD = NW * K_CHUNKS * CHUNK
ACC_ROWS = 10240
STRIPE = ACC_ROWS // NS


def _sc_segment_sum(x, src3, dst3):
    mesh = plsc.VectorSubcoreMesh(core_axis_name="c", subcore_axis_name="s")

    @functools.partial(
        pl.kernel,
        mesh=mesh,
        out_type=jax.ShapeDtypeStruct((NC, ACC_ROWS, D_FEAT), jnp.float32),
        scratch_types=[
            pltpu.VMEM((K_CHUNKS, CHUNK), jnp.int32),
            pltpu.VMEM((K_CHUNKS, CHUNK), jnp.int32),
            pltpu.VMEM((CHUNK, D_FEAT), jnp.float32),
            pltpu.VMEM_SHARED((ACC_ROWS, D_FEAT), jnp.float32),
            pltpu.SemaphoreType.DMA,
        ],
    )
    def k(x_hbm, src_hbm, dst_hbm, out_hbm, sidx, didx, rows, acc, sem):
        c = lax.axis_index("c")
        s = lax.axis_index("s")
        wid = s * NC + c

        @pl.loop(0, CHUNK)
        def _(r):
            @pl.loop(0, D_FEAT, step=LANES)
            def _(col):
                rows.at[pl.ds(r, 1), pl.ds(col, LANES)][...] = jnp.zeros(
                    (1, LANES), jnp.float32)

        base = s * STRIPE
        for b in range(STRIPE // CHUNK):
            pltpu.sync_copy(rows, acc.at[pl.ds(base + b * CHUNK, CHUNK)])

        plsc.subcore_barrier()

        pltpu.sync_copy(src_hbm.at[wid], sidx)
        pltpu.sync_copy(dst_hbm.at[wid], didx)

        @pl.loop(0, K_CHUNKS)
        def _(j):
            pltpu.async_copy(x_hbm.at[sidx.at[j]], rows, sem).wait()
            pltpu.sync_copy(rows, acc.at[didx.at[j]], add=True)

        plsc.subcore_barrier()

        pltpu.sync_copy(acc.at[pl.ds(base, STRIPE)],
                        out_hbm.at[c, pl.ds(base, STRIPE)])

    return k(x, src3, dst3)


def _combine(parts, W):
    blk = 1000

    def body(p_ref, w_ref, o_ref):
        o_ref[...] = (p_ref[0] + p_ref[1]) * w_ref[...]

    return pl.pallas_call(
        body,
        grid=(N_NODES // blk,),
        in_specs=[
            pl.BlockSpec((NC, blk, D_FEAT), lambda i: (0, i, 0)),
            pl.BlockSpec((1, D_FEAT), lambda i: (0, 0)),
        ],
        out_specs=pl.BlockSpec((blk, D_FEAT), lambda i: (i, 0)),
        out_shape=jax.ShapeDtypeStruct((N_NODES, D_FEAT), jnp.float32),
    )(parts, W)


def kernel(x, edge_index, W):
    src = edge_index[0]
    dst = edge_index[1]
    pad = E_PAD - N_EDGES
    pad_src = (jnp.arange(pad, dtype=jnp.int32) * 97) % N_NODES
    pad_dst = N_NODES + (jnp.arange(pad, dtype=jnp.int32) % (ACC_ROWS - N_NODES))
    src3 = jnp.concatenate([src, pad_src]).reshape(NW, K_CHUNKS, CHUNK)
    dst3 = jnp.concatenate([dst, pad_dst]).reshape(NW, K_CHUNKS, CHUNK)
    parts = _sc_segment_sum(x, src3, dst3)
    return _combine(parts, W)


# speedup vs baseline: 4.8242x; 1.4842x over previous
import functools

import jax
import jax.numpy as jnp
from jax import lax
from jax.experimental import pallas as pl
from jax.experimental.pallas import tpu as pltpu
from jax.experimental.pallas import tpu_sc as plsc

N_NODES = 10000
N_EDGES = 320000
D_FEAT = 128

NC = 2
NS = 16
NW = NC * NS
LANES = 16

CHUNK = 128
K_CHUNKS = -(-N_EDGES // (NW * CHUNK))
K_CHUNKS += K_CHUNKS % 2
G = 4
NG = K_CHUNKS // G
K_IDX = K_CHUNKS + 2 * G
E_PAD = NW * K_CHUNKS * CHUNK
ACC_ROWS = 10240
JUNK_ROWS = ACC_ROWS - N_NODES
STRIPE = ACC_ROWS // NS


def _sc_segment_sum(x, src3, dst3):
    mesh = plsc.VectorSubcoreMesh(core_axis_name="c", subcore_axis_name="s")

    @functools.partial(
        pl.kernel,
        mesh=mesh,
        out_type=jax.ShapeDtypeStruct((NC, ACC_ROWS, D_FEAT), jnp.float32),
        scratch_types=[
            pltpu.VMEM((K_IDX, CHUNK), jnp.int32),
            pltpu.VMEM((G, CHUNK), jnp.int32),
            pltpu.VMEM((G, CHUNK), jnp.int32),
            pltpu.VMEM((CHUNK, D_FEAT), jnp.float32),
            pltpu.VMEM((CHUNK, D_FEAT), jnp.float32),
            pltpu.VMEM_SHARED((ACC_ROWS, D_FEAT), jnp.float32),
            pltpu.SemaphoreType.DMA,
            pltpu.SemaphoreType.DMA,
            pltpu.SemaphoreType.DMA,
            pltpu.SemaphoreType.DMA,
        ],
    )
    def k(x_hbm, src_hbm, dst_hbm, out_hbm, sidx, dg0, dg1, rows0, rows1,
          acc, g0, g1, i0, i1):
        c = lax.axis_index("c")
        s = lax.axis_index("s")
        wid = s * NC + c

        def wait_dst(dg, isem):
            pltpu.make_async_copy(
                dst_hbm.at[wid, pl.ds(0, G)], dg, isem).wait()

        h_src = pltpu.async_copy(src_hbm.at[wid], sidx, g0)
        pltpu.async_copy(dst_hbm.at[wid, pl.ds(0, G)], dg0, i0)
        pltpu.async_copy(dst_hbm.at[wid, pl.ds(G, G)], dg1, i1)

        @pl.loop(0, CHUNK)
        def _(r):
            @pl.loop(0, D_FEAT, step=LANES)
            def _(col):
                rows0.at[pl.ds(r, 1), pl.ds(col, LANES)][...] = jnp.zeros(
                    (1, LANES), jnp.float32)

        base = s * STRIPE
        for b in range(STRIPE // CHUNK):
            pltpu.sync_copy(rows0, acc.at[pl.ds(base + b * CHUNK, CHUNK)])

        h_src.wait()
        plsc.subcore_barrier()

        pltpu.async_copy(x_hbm.at[sidx.at[0]], rows0, g0)

        @pl.loop(0, NG, step=2)
        def _(p):
            for gslot, dg, isem in ((0, dg0, i0), (1, dg1, i1)):
                grp = p + gslot
                jb = grp * G
                wait_dst(dg, isem)
                for t in range(G):
                    ch = jb + t
                    rb, rnb = (rows0, rows1) if t % 2 == 0 else (rows1, rows0)
                    sb, snb = (g0, g1) if t % 2 == 0 else (g1, g0)
                    pltpu.async_copy(x_hbm.at[sidx.at[ch + 1]], rnb, snb)
                    pltpu.make_async_copy(x_hbm.at[sidx.at[ch]], rb, sb).wait()
                    pltpu.sync_copy(rb, acc.at[dg.at[t]], add=True)
                pltpu.async_copy(
                    dst_hbm.at[wid, pl.ds((grp + 2) * G, G)], dg, isem)

        wait_dst(dg0, i0)
        wait_dst(dg1, i1)
        pltpu.make_async_copy(x_hbm.at[sidx.at[0]], rows0, g0).wait()

        plsc.subcore_barrier()

        pltpu.sync_copy(acc.at[pl.ds(base, STRIPE)],
                        out_hbm.at[c, pl.ds(base, STRIPE)])

    return k(x, src3, dst3)


def _combine(parts, W):
    blk = 1000

    def body(p_ref, w_ref, o_ref):
        o_ref[...] = (p_ref[0] + p_ref[1]) * w_ref[...]

    return pl.pallas_call(
        body,
        grid=(N_NODES // blk,),
        in_specs=[
            pl.BlockSpec((NC, blk, D_FEAT), lambda i: (0, i, 0)),
            pl.BlockSpec((1, D_FEAT), lambda i: (0, 0)),
        ],
        out_specs=pl.BlockSpec((blk, D_FEAT), lambda i: (i, 0)),
        out_shape=jax.ShapeDtypeStruct((N_NODES, D_FEAT), jnp.float32),
    )(parts, W)


def kernel(x, edge_index, W):
    src = edge_index[0]
    dst = edge_index[1]
    pad = E_PAD - N_EDGES
    pad_src = (jnp.arange(pad, dtype=jnp.int32) * 97) % N_NODES
    pad_dst = N_NODES + (jnp.arange(pad, dtype=jnp.int32) % JUNK_ROWS)
    ovr = K_IDX - K_CHUNKS
    n_ovr = NW * ovr * CHUNK
    ovr_src = ((jnp.arange(n_ovr, dtype=jnp.int32) * 89) % N_NODES).reshape(
        NW, ovr, CHUNK)
    ovr_dst = (N_NODES + jnp.arange(n_ovr, dtype=jnp.int32) % JUNK_ROWS
               ).reshape(NW, ovr, CHUNK)
    src3 = jnp.concatenate(
        [jnp.concatenate([src, pad_src]).reshape(NW, K_CHUNKS, CHUNK),
         ovr_src], axis=1)
    dst3 = jnp.concatenate(
        [jnp.concatenate([dst, pad_dst]).reshape(NW, K_CHUNKS, CHUNK),
         ovr_dst], axis=1)
    parts = _sc_segment_sum(x, src3, dst3)
    return _combine(parts, W)
